# baseline (speedup 1.0000x reference)
import functools

import jax
import jax.numpy as jnp
from jax import lax
from jax.experimental import pallas as pl
from jax.experimental.pallas import tpu as pltpu
from jax.experimental.pallas import tpu_sc as plsc

D = 128
N = 10000
NP = 10240
E = 320000
NC = 2
NS = 16
NW = NC * NS
CH = 128
NCHW = 80
E_PAD = NW * NCHW * CH
REAL_CROWS = E // CH
ROWS_PER_TILE = NP // NS


def _mesh():
    return plsc.VectorSubcoreMesh(core_axis_name="c", subcore_axis_name="s")


def _wid():
    return lax.axis_index("s") * NC + lax.axis_index("c")


def _sc_gather_rows(table, idx, ch):
    (b,) = idx.shape
    _, d = table.shape
    bpw = b // NW
    nch = bpw // ch

    @functools.partial(
        pl.kernel,
        out_type=jax.ShapeDtypeStruct((b, d), jnp.float32),
        mesh=_mesh(),
        scratch_types=[
            pltpu.VMEM((ch,), jnp.int32),
            pltpu.VMEM((ch, d), jnp.float32),
            pltpu.SemaphoreType.DMA,
        ],
    )
    def k(table_h, idx_h, out_h, idx_v, rows_v, sem):
        base = _wid() * bpw

        def body(j, carry):
            off = pl.multiple_of(base + j * ch, 8)
            pltpu.sync_copy(idx_h.at[pl.ds(off, ch)], idx_v)
            pltpu.async_copy(table_h.at[idx_v], rows_v, sem).wait()
            pltpu.sync_copy(rows_v, out_h.at[pl.ds(off, ch)])
            return carry

        lax.fori_loop(0, nch, body, 0)

    return k(table, idx)


def _sc_msgpass(x, ea, src, dst, rel, zblk):
    MCH = 64
    MNCH = E_PAD // (NW * MCH)
    EW = MNCH * MCH

    @functools.partial(
        pl.kernel,
        out_type=jax.ShapeDtypeStruct((NC, NP, D), jnp.float32),
        mesh=_mesh(),
        scratch_types=[
            pltpu.VMEM((MCH,), jnp.int32),
            pltpu.VMEM((MCH,), jnp.int32),
            pltpu.VMEM((MCH,), jnp.int32),
            pltpu.VMEM((MCH,), jnp.int32),
            pltpu.VMEM((MCH,), jnp.int32),
            pltpu.VMEM((MCH,), jnp.int32),
            pltpu.VMEM((MCH, D), jnp.float32),
            pltpu.VMEM((MCH, D), jnp.float32),
            pltpu.VMEM((MCH, D), jnp.float32),
            pltpu.VMEM((MCH, D), jnp.float32),
            pltpu.VMEM_SHARED((NP, D), jnp.float32),
            pltpu.SemaphoreType.DMA,
            pltpu.SemaphoreType.DMA,
            pltpu.SemaphoreType.DMA,
            pltpu.SemaphoreType.DMA,
            pltpu.SemaphoreType.DMA,
            pltpu.SemaphoreType.DMA,
        ],
    )
    def k(x_h, ea_h, src_h, dst_h, rel_h, z_h, out_h,
          sia, sib, ria, rib, dia, dib, xsa, xsb, eaa, eab, agg,
          sxa, sxb, sea, seb, sda, sdb):
        cid = lax.axis_index("c")
        sid = lax.axis_index("s")
        base = (sid * NC + cid) * EW

        def zbody(t, carry):
            r0 = pl.multiple_of(sid * ROWS_PER_TILE + t * CH, 8)
            pltpu.sync_copy(z_h, agg.at[pl.ds(r0, CH)])
            return carry

        lax.fori_loop(0, ROWS_PER_TILE // CH, zbody, 0)
        plsc.subcore_barrier()

        def issue_idx(j, si_, ri_, di_, sd):
            off = pl.multiple_of(base + j * MCH, 8)
            pltpu.async_copy(src_h.at[pl.ds(off, MCH)], si_, sd)
            pltpu.async_copy(rel_h.at[pl.ds(off, MCH)], ri_, sd)
            pltpu.async_copy(dst_h.at[pl.ds(off, MCH)], di_, sd)

        def wait_idx(j, si_, ri_, di_, sd):
            off = pl.multiple_of(base + j * MCH, 8)
            pltpu.make_async_copy(src_h.at[pl.ds(off, MCH)], si_, sd).wait()
            pltpu.make_async_copy(rel_h.at[pl.ds(off, MCH)], ri_, sd).wait()
            pltpu.make_async_copy(dst_h.at[pl.ds(off, MCH)], di_, sd).wait()

        def issue_gath(si_, ri_, xs_, ea_, sx, se):
            pltpu.async_copy(x_h.at[si_], xs_, sx)
            pltpu.async_copy(ea_h.at[ri_], ea_, se)

        def wait_gath(si_, ri_, xs_, ea_, sx, se):
            pltpu.make_async_copy(x_h.at[si_], xs_, sx).wait()
            pltpu.make_async_copy(ea_h.at[ri_], ea_, se).wait()

        def compute_scatter(xs_, ea_, di_):
            def cbody(i, carry):
                for jj in range(D // 16):
                    sl = pl.ds(jj * 16, 16)
                    xs_[i, sl] = jnp.maximum(xs_[i, sl] + ea_[i, sl], 0.0)
                return carry

            lax.fori_loop(0, MCH, cbody, 0)
            pltpu.sync_copy(xs_, agg.at[di_], add=True)

        issue_idx(0, sia, ria, dia, sda)
        issue_idx(1, sib, rib, dib, sdb)
        wait_idx(0, sia, ria, dia, sda)
        issue_gath(sia, ria, xsa, eaa, sxa, sea)
        last_t = MNCH // 2 - 1

        def ebody(t, carry):
            c0 = 2 * t
            c1 = 2 * t + 1
            wait_idx(c1, sib, rib, dib, sdb)
            issue_gath(sib, rib, xsb, eab, sxb, seb)
            wait_gath(sia, ria, xsa, eaa, sxa, sea)
            compute_scatter(xsa, eaa, dia)

            @pl.when(t < last_t)
            def _():
                issue_idx(c0 + 2, sia, ria, dia, sda)

            wait_gath(sib, rib, xsb, eab, sxb, seb)

            @pl.when(t < last_t)
            def _():
                wait_idx(c0 + 2, sia, ria, dia, sda)
                issue_gath(sia, ria, xsa, eaa, sxa, sea)

            compute_scatter(xsb, eab, dib)

            @pl.when(t < last_t)
            def _():
                issue_idx(c1 + 2, sib, rib, dib, sdb)

            return carry

        lax.fori_loop(0, MNCH // 2, ebody, 0)

        plsc.subcore_barrier()

        def obody(t, carry):
            r0 = pl.multiple_of(sid * ROWS_PER_TILE + t * CH, 8)
            pltpu.sync_copy(agg.at[pl.ds(r0, CH)], out_h.at[cid, pl.ds(r0, CH)])
            return carry

        lax.fori_loop(0, ROWS_PER_TILE // CH, obody, 0)

    return k(x, ea, src, dst, rel, zblk)


def _sc_edge_head(s_head, d_head, src, dst):
    k64 = 64
    EW = NCHW * CH

    @functools.partial(
        pl.kernel,
        out_type=jax.ShapeDtypeStruct((E, k64), jnp.float32),
        mesh=_mesh(),
        scratch_types=[
            pltpu.VMEM((EW,), jnp.int32),
            pltpu.VMEM((EW,), jnp.int32),
            pltpu.VMEM((CH, k64), jnp.float32),
            pltpu.VMEM((CH, k64), jnp.float32),
            pltpu.VMEM((CH, k64), jnp.float32),
            pltpu.VMEM((CH, k64), jnp.float32),
            pltpu.SemaphoreType.DMA,
            pltpu.SemaphoreType.DMA,
            pltpu.SemaphoreType.DMA,
            pltpu.SemaphoreType.DMA,
        ],
        compiler_params=pltpu.CompilerParams(use_tc_tiling_on_sc=False),
    )
    def k(s_h, d_h, src_h, dst_h, out_h,
          si, di, sva, svb, dva, dvb, ssa, ssb, sda, sdb):
        crow0 = _wid() * NCHW
        base = crow0 * CH

        pltpu.sync_copy(src_h.at[pl.ds(base, EW)], si)
        pltpu.sync_copy(dst_h.at[pl.ds(base, EW)], di)

        def issue(j, sv_, dv_, ss, sd):
            iw = pl.multiple_of(j * CH, 8)
            pltpu.async_copy(s_h.at[si.at[pl.ds(iw, CH)]], sv_, ss)
            pltpu.async_copy(d_h.at[di.at[pl.ds(iw, CH)]], dv_, sd)

        def wait(j, sv_, dv_, ss, sd):
            iw = pl.multiple_of(j * CH, 8)
            pltpu.make_async_copy(s_h.at[si.at[pl.ds(iw, CH)]], sv_, ss).wait()
            pltpu.make_async_copy(d_h.at[di.at[pl.ds(iw, CH)]], dv_, sd).wait()

        def compute_store(j, sv_, dv_):
            def cbody(i, carry):
                for jj in range(k64 // 16):
                    sl = pl.ds(jj * 16, 16)
                    sv_[i, sl] = sv_[i, sl] + dv_[i, sl]
                return carry

            lax.fori_loop(0, CH, cbody, 0)

            @pl.when(crow0 + j < REAL_CROWS)
            def _():
                off = pl.multiple_of((crow0 + j) * CH, 8)
                pltpu.sync_copy(sv_, out_h.at[pl.ds(off, CH)])

        issue(0, sva, dva, ssa, sda)

        def ebody(t, carry):
            c0 = 2 * t
            c1 = 2 * t + 1
            issue(c1, svb, dvb, ssb, sdb)
            wait(c0, sva, dva, ssa, sda)
            compute_store(c0, sva, dva)

            @pl.when(t < NCHW // 2 - 1)
            def _():
                issue(c0 + 2, sva, dva, ssa, sda)

            wait(c1, svb, dvb, ssb, sdb)
            compute_store(c1, svb, dvb)
            return carry

        lax.fori_loop(0, NCHW // 2, ebody, 0)

    return k(s_head, d_head, src, dst)


def _tc_dense(x, agg, wt, b, do_norm, do_relu, blk, row_off=0):
    bb, d = x.shape
    kk = wt.shape[1]
    has_agg = agg is not None
    nrows = bb - row_off * blk

    def body(*refs):
        xv = refs[0][...]
        if has_agg:
            av = refs[1][...]
            xv = xv + av[0] + av[1]
        iw = 1 + int(has_agg)
        if do_norm:
            s = jnp.sum(xv * xv, axis=1, keepdims=True)
            xv = xv / jnp.maximum(jnp.sqrt(s), 1e-12)
        y = jnp.dot(xv, refs[iw][...], preferred_element_type=jnp.float32)
        y = y + refs[iw + 1][...]
        if do_relu:
            y = jnp.maximum(y, 0.0)
        refs[iw + 2][...] = y

    in_specs = [pl.BlockSpec((blk, d), lambda i: (i + row_off, 0))]
    if has_agg:
        in_specs.append(pl.BlockSpec((NC, blk, d), lambda i: (0, i + row_off, 0)))
    in_specs += [
        pl.BlockSpec((d, kk), lambda i: (0, 0)),
        pl.BlockSpec((1, kk), lambda i: (0, 0)),
    ]
    args = (x, agg, wt, b) if has_agg else (x, wt, b)
    return pl.pallas_call(
        body,
        grid=(nrows // blk,),
        in_specs=in_specs,
        out_specs=pl.BlockSpec((blk, kk), lambda i: (i, 0)),
        out_shape=jax.ShapeDtypeStruct((nrows, kk), jnp.float32),
    )(*args)


def _tc_heads(x2, ws, bs, wd, wb, bb_):
    bb, d = x2.shape
    blk = 512

    def body(x_ref, ws_ref, bs_ref, wd_ref, wb_ref, bb_ref, s_ref, d_ref, b_ref):
        xv = x_ref[...]
        s_ref[...] = jnp.dot(xv, ws_ref[...], preferred_element_type=jnp.float32) + bs_ref[...]
        d_ref[...] = jnp.dot(xv, wd_ref[...], preferred_element_type=jnp.float32)
        b_ref[...] = jnp.dot(xv, wb_ref[...], preferred_element_type=jnp.float32) + bb_ref[...]

    return pl.pallas_call(
        body,
        grid=(bb // blk,),
        in_specs=[
            pl.BlockSpec((blk, d), lambda i: (i, 0)),
            pl.BlockSpec((d, 64), lambda i: (0, 0)),
            pl.BlockSpec((1, 64), lambda i: (0, 0)),
            pl.BlockSpec((d, 64), lambda i: (0, 0)),
            pl.BlockSpec((d, 8), lambda i: (0, 0)),
            pl.BlockSpec((1, 8), lambda i: (0, 0)),
        ],
        out_specs=[
            pl.BlockSpec((blk, 64), lambda i: (i, 0)),
            pl.BlockSpec((blk, 64), lambda i: (i, 0)),
            pl.BlockSpec((blk, 8), lambda i: (i, 0)),
        ],
        out_shape=[
            jax.ShapeDtypeStruct((bb, 64), jnp.float32),
            jax.ShapeDtypeStruct((bb, 64), jnp.float32),
            jax.ShapeDtypeStruct((bb, 8), jnp.float32),
        ],
    )(x2, ws, bs, wd, wb, bb_)


def kernel(node_ids, rel_ids, center_mol_idx, non_molecule_node_ids, edge_index,
           node_emb_table, rel_emb_table, lin_W, lin_b,
           conv1_W, conv1_b, conv2_W, conv2_b,
           ec_W, ec_b, mp_W, mp_b, nc_W, nc_b, bp_W, bp_b):
    f32 = jnp.float32
    i32 = jnp.int32
    node_ids = node_ids.astype(i32)
    rel_ids = rel_ids.astype(i32)
    src = edge_index[0].astype(i32)
    dst = edge_index[1].astype(i32)

    npad = E_PAD - E
    src_p = jnp.concatenate([src, jnp.zeros((npad,), i32)])
    rel_p = jnp.concatenate([rel_ids, jnp.zeros((npad,), i32)])
    dump = N + (jnp.arange(npad, dtype=i32) % (NP - N))
    dst_p = jnp.concatenate([dst, dump])

    ids_pad = jnp.concatenate([node_ids, jnp.zeros((NP - N,), i32)])
    rows = _sc_gather_rows(node_emb_table.astype(f32), ids_pad, 64)

    lin_bt = lin_b.reshape(1, -1)
    x0 = _tc_dense(rows, None, lin_W.T, lin_bt, True, True, 512)
    ea_u = _tc_dense(rel_emb_table.astype(f32), None, lin_W.T, lin_bt, True, True, 64)

    zblk = jnp.zeros((CH, D), f32)
    agg1 = _sc_msgpass(x0, ea_u, src_p, dst_p, rel_p, zblk)
    x1 = _tc_dense(x0, agg1, conv1_W.T, conv1_b.reshape(1, -1), False, True, 512)
    agg2 = _sc_msgpass(x1, ea_u, src_p, dst_p, rel_p, zblk)
    x2 = _tc_dense(x1, agg2, conv2_W.T, conv2_b.reshape(1, -1), False, False, 512)

    wb = jnp.pad(bp_W.T, ((0, 0), (0, 7)))
    bb_ = jnp.pad(bp_b, (0, 7)).reshape(1, -1)
    s_head, d_head, bpad = _tc_heads(
        x2, ec_W[:, :D].T, ec_b.reshape(1, -1), ec_W[:, D:].T, wb, bb_)
    binary_pred = bpad[:N, :1]

    edge_class = _sc_edge_head(s_head, d_head, src_p, dst_p)

    gidx = jnp.concatenate([center_mol_idx.astype(i32),
                            non_molecule_node_ids.astype(i32)])
    xg = _sc_gather_rows(x2, gidx, 96)
    mp_wt = jnp.pad(mp_W.T, ((0, 0), (0, 28)))
    mp_bp = jnp.pad(mp_b, (0, 28)).reshape(1, -1)
    motif_pred = _tc_dense(xg, None, mp_wt, mp_bp, False, False, 512)[:1024, :100]
    nc_wt = jnp.pad(nc_W.T, ((0, 0), (0, 113)))
    nc_bp = jnp.pad(nc_b, (0, 113)).reshape(1, -1)
    node_class = _tc_dense(xg, None, nc_wt, nc_bp, False, False, 512, row_off=2)[:, :15]

    return (edge_class, motif_pred, node_class, binary_pred)


# speedup vs baseline: 1.0103x; 1.0103x over previous
import functools

import jax
import jax.numpy as jnp
from jax import lax
from jax.experimental import pallas as pl
from jax.experimental.pallas import tpu as pltpu
from jax.experimental.pallas import tpu_sc as plsc

D = 128
N = 10000
NP = 10240
E = 320000
NC = 2
NS = 16
NW = NC * NS
CH = 128
NCHW = 80
E_PAD = NW * NCHW * CH
REAL_CROWS = E // CH
ROWS_PER_TILE = NP // NS


def _mesh():
    return plsc.VectorSubcoreMesh(core_axis_name="c", subcore_axis_name="s")


def _wid():
    return lax.axis_index("s") * NC + lax.axis_index("c")


def _sc_gather_rows(table, idx, ch):
    (b,) = idx.shape
    _, d = table.shape
    bpw = b // NW
    nch = bpw // ch

    @functools.partial(
        pl.kernel,
        out_type=jax.ShapeDtypeStruct((b, d), jnp.float32),
        mesh=_mesh(),
        scratch_types=[
            pltpu.VMEM((ch,), jnp.int32),
            pltpu.VMEM((ch, d), jnp.float32),
            pltpu.SemaphoreType.DMA,
        ],
    )
    def k(table_h, idx_h, out_h, idx_v, rows_v, sem):
        base = _wid() * bpw

        def body(j, carry):
            off = pl.multiple_of(base + j * ch, 8)
            pltpu.sync_copy(idx_h.at[pl.ds(off, ch)], idx_v)
            pltpu.async_copy(table_h.at[idx_v], rows_v, sem).wait()
            pltpu.sync_copy(rows_v, out_h.at[pl.ds(off, ch)])
            return carry

        lax.fori_loop(0, nch, body, 0)

    return k(table, idx)


def _sc_msgpass(x, ea, src, dst, rel, zblk):
    MCH = 64
    MNCH = E_PAD // (NW * MCH)
    EW = MNCH * MCH

    @functools.partial(
        pl.kernel,
        out_type=jax.ShapeDtypeStruct((NC, NP, D), jnp.float32),
        mesh=_mesh(),
        scratch_types=[
            pltpu.VMEM((MCH,), jnp.int32),
            pltpu.VMEM((MCH,), jnp.int32),
            pltpu.VMEM((MCH,), jnp.int32),
            pltpu.VMEM((MCH,), jnp.int32),
            pltpu.VMEM((MCH,), jnp.int32),
            pltpu.VMEM((MCH,), jnp.int32),
            pltpu.VMEM((MCH, D), jnp.float32),
            pltpu.VMEM((MCH, D), jnp.float32),
            pltpu.VMEM((MCH, D), jnp.float32),
            pltpu.VMEM((MCH, D), jnp.float32),
            pltpu.VMEM_SHARED((NP, D), jnp.float32),
            pltpu.SemaphoreType.DMA,
            pltpu.SemaphoreType.DMA,
            pltpu.SemaphoreType.DMA,
            pltpu.SemaphoreType.DMA,
            pltpu.SemaphoreType.DMA,
            pltpu.SemaphoreType.DMA,
        ],
    )
    def k(x_h, ea_h, src_h, dst_h, rel_h, z_h, out_h,
          sia, sib, ria, rib, dia, dib, xsa, xsb, eaa, eab, agg,
          sxa, sxb, sea, seb, sda, sdb):
        cid = lax.axis_index("c")
        sid = lax.axis_index("s")
        base = (sid * NC + cid) * EW

        def zbody(t, carry):
            r0 = pl.multiple_of(sid * ROWS_PER_TILE + t * CH, 8)
            pltpu.sync_copy(z_h, agg.at[pl.ds(r0, CH)])
            return carry

        lax.fori_loop(0, ROWS_PER_TILE // CH, zbody, 0)
        plsc.subcore_barrier()

        def issue_idx(j, si_, ri_, di_, sd):
            off = pl.multiple_of(base + j * MCH, 8)
            pltpu.async_copy(src_h.at[pl.ds(off, MCH)], si_, sd)
            pltpu.async_copy(rel_h.at[pl.ds(off, MCH)], ri_, sd)
            pltpu.async_copy(dst_h.at[pl.ds(off, MCH)], di_, sd)

        def wait_idx(j, si_, ri_, di_, sd):
            off = pl.multiple_of(base + j * MCH, 8)
            pltpu.make_async_copy(src_h.at[pl.ds(off, MCH)], si_, sd).wait()
            pltpu.make_async_copy(rel_h.at[pl.ds(off, MCH)], ri_, sd).wait()
            pltpu.make_async_copy(dst_h.at[pl.ds(off, MCH)], di_, sd).wait()

        def issue_gath(si_, ri_, xs_, ea_, sx, se):
            pltpu.async_copy(x_h.at[si_], xs_, sx)
            pltpu.async_copy(ea_h.at[ri_], ea_, se)

        def wait_gath(si_, ri_, xs_, ea_, sx, se):
            pltpu.make_async_copy(x_h.at[si_], xs_, sx).wait()
            pltpu.make_async_copy(ea_h.at[ri_], ea_, se).wait()

        def compute_scatter(xs_, ea_, di_):
            def cbody(i, carry):
                for jj in range(D // 16):
                    sl = pl.ds(jj * 16, 16)
                    xs_[i, sl] = jnp.maximum(xs_[i, sl] + ea_[i, sl], 0.0)
                return carry

            lax.fori_loop(0, MCH, cbody, 0)
            pltpu.sync_copy(xs_, agg.at[di_], add=True)

        issue_idx(0, sia, ria, dia, sda)
        issue_idx(1, sib, rib, dib, sdb)
        wait_idx(0, sia, ria, dia, sda)
        issue_gath(sia, ria, xsa, eaa, sxa, sea)
        last_t = MNCH // 2 - 1

        def ebody(t, carry):
            c0 = 2 * t
            c1 = 2 * t + 1
            wait_idx(c1, sib, rib, dib, sdb)
            issue_gath(sib, rib, xsb, eab, sxb, seb)
            wait_gath(sia, ria, xsa, eaa, sxa, sea)
            compute_scatter(xsa, eaa, dia)

            @pl.when(t < last_t)
            def _():
                issue_idx(c0 + 2, sia, ria, dia, sda)

            wait_gath(sib, rib, xsb, eab, sxb, seb)

            @pl.when(t < last_t)
            def _():
                wait_idx(c0 + 2, sia, ria, dia, sda)
                issue_gath(sia, ria, xsa, eaa, sxa, sea)

            compute_scatter(xsb, eab, dib)

            @pl.when(t < last_t)
            def _():
                issue_idx(c1 + 2, sib, rib, dib, sdb)

            return carry

        lax.fori_loop(0, MNCH // 2, ebody, 0)

        plsc.subcore_barrier()

        def obody(t, carry):
            r0 = pl.multiple_of(sid * ROWS_PER_TILE + t * CH, 8)
            pltpu.sync_copy(agg.at[pl.ds(r0, CH)], out_h.at[cid, pl.ds(r0, CH)])
            return carry

        lax.fori_loop(0, ROWS_PER_TILE // CH, obody, 0)

    return k(x, ea, src, dst, rel, zblk)


def _sc_edge_head(s_head, d_head, src, dst):
    k64 = 64
    EW = NCHW * CH

    @functools.partial(
        pl.kernel,
        out_type=jax.ShapeDtypeStruct((E, k64), jnp.float32),
        mesh=_mesh(),
        scratch_types=[
            pltpu.VMEM((EW,), jnp.int32),
            pltpu.VMEM((EW,), jnp.int32),
            pltpu.VMEM((CH, k64), jnp.float32),
            pltpu.VMEM((CH, k64), jnp.float32),
            pltpu.VMEM((CH, k64), jnp.float32),
            pltpu.VMEM((CH, k64), jnp.float32),
            pltpu.SemaphoreType.DMA,
            pltpu.SemaphoreType.DMA,
            pltpu.SemaphoreType.DMA,
            pltpu.SemaphoreType.DMA,
            pltpu.SemaphoreType.DMA,
            pltpu.SemaphoreType.DMA,
        ],
        compiler_params=pltpu.CompilerParams(use_tc_tiling_on_sc=False),
    )
    def k(s_h, d_h, src_h, dst_h, out_h,
          si, di, sva, svb, dva, dvb, ssa, ssb, sda, sdb, swa, swb):
        crow0 = _wid() * NCHW
        base = crow0 * CH

        pltpu.sync_copy(src_h.at[pl.ds(base, EW)], si)
        pltpu.sync_copy(dst_h.at[pl.ds(base, EW)], di)

        def issue(j, sv_, dv_, ss, sd):
            iw = pl.multiple_of(j * CH, 8)
            pltpu.async_copy(s_h.at[si.at[pl.ds(iw, CH)]], sv_, ss)
            pltpu.async_copy(d_h.at[di.at[pl.ds(iw, CH)]], dv_, sd)

        def wait(j, sv_, dv_, ss, sd):
            iw = pl.multiple_of(j * CH, 8)
            pltpu.make_async_copy(s_h.at[si.at[pl.ds(iw, CH)]], sv_, ss).wait()
            pltpu.make_async_copy(d_h.at[di.at[pl.ds(iw, CH)]], dv_, sd).wait()

        def compute_store(j, sv_, dv_, sw):
            def cbody(i, carry):
                for jj in range(k64 // 16):
                    sl = pl.ds(jj * 16, 16)
                    sv_[i, sl] = sv_[i, sl] + dv_[i, sl]
                return carry

            lax.fori_loop(0, CH, cbody, 0)

            @pl.when(crow0 + j < REAL_CROWS)
            def _():
                off = pl.multiple_of((crow0 + j) * CH, 8)
                pltpu.async_copy(sv_, out_h.at[pl.ds(off, CH)], sw)

        def wait_store(j, sv_, sw):
            @pl.when(crow0 + j < REAL_CROWS)
            def _():
                off = pl.multiple_of((crow0 + j) * CH, 8)
                pltpu.make_async_copy(sv_, out_h.at[pl.ds(off, CH)], sw).wait()

        issue(0, sva, dva, ssa, sda)
        last_t = NCHW // 2 - 1

        def ebody(t, carry):
            c0 = 2 * t
            c1 = 2 * t + 1

            @pl.when(t > 0)
            def _():
                wait_store(c1 - 2, svb, swb)

            issue(c1, svb, dvb, ssb, sdb)
            wait(c0, sva, dva, ssa, sda)
            compute_store(c0, sva, dva, swa)

            @pl.when(t < last_t)
            def _():
                wait_store(c0, sva, swa)
                issue(c0 + 2, sva, dva, ssa, sda)

            wait(c1, svb, dvb, ssb, sdb)
            compute_store(c1, svb, dvb, swb)
            return carry

        lax.fori_loop(0, NCHW // 2, ebody, 0)
        wait_store(NCHW - 2, sva, swa)
        wait_store(NCHW - 1, svb, swb)

    return k(s_head, d_head, src, dst)


def _tc_dense(x, agg, wt, b, do_norm, do_relu, blk, row_off=0, mask_pad=False):
    bb, d = x.shape
    kk = wt.shape[1]
    has_agg = agg is not None
    nrows = bb - row_off * blk

    def body(*refs):
        xv = refs[0][...]
        if has_agg:
            av = refs[1][...]
            xv = xv + av[0] + av[1]
        iw = 1 + int(has_agg)
        if do_norm:
            s = jnp.sum(xv * xv, axis=1, keepdims=True)
            xv = xv / jnp.maximum(jnp.sqrt(s), 1e-12)
        y = jnp.dot(xv, refs[iw][...], preferred_element_type=jnp.float32)
        y = y + refs[iw + 1][...]
        if do_relu:
            y = jnp.maximum(y, 0.0)
        if mask_pad:
            rows = (jax.lax.broadcasted_iota(jnp.int32, (blk, kk), 0)
                    + pl.program_id(0) * blk)
            y = jnp.where(rows < N, y, -1e9)
        refs[iw + 2][...] = y

    in_specs = [pl.BlockSpec((blk, d), lambda i: (i + row_off, 0))]
    if has_agg:
        in_specs.append(pl.BlockSpec((NC, blk, d), lambda i: (0, i + row_off, 0)))
    in_specs += [
        pl.BlockSpec((d, kk), lambda i: (0, 0)),
        pl.BlockSpec((1, kk), lambda i: (0, 0)),
    ]
    args = (x, agg, wt, b) if has_agg else (x, wt, b)
    return pl.pallas_call(
        body,
        grid=(nrows // blk,),
        in_specs=in_specs,
        out_specs=pl.BlockSpec((blk, kk), lambda i: (i, 0)),
        out_shape=jax.ShapeDtypeStruct((nrows, kk), jnp.float32),
    )(*args)


def _tc_heads(x2, ws, bs, wd, wb, bb_):
    bb, d = x2.shape
    blk = 512

    def body(x_ref, ws_ref, bs_ref, wd_ref, wb_ref, bb_ref, s_ref, d_ref, b_ref):
        xv = x_ref[...]
        s_ref[...] = jnp.dot(xv, ws_ref[...], preferred_element_type=jnp.float32) + bs_ref[...]
        d_ref[...] = jnp.dot(xv, wd_ref[...], preferred_element_type=jnp.float32)
        b_ref[...] = jnp.dot(xv, wb_ref[...], preferred_element_type=jnp.float32) + bb_ref[...]

    return pl.pallas_call(
        body,
        grid=(bb // blk,),
        in_specs=[
            pl.BlockSpec((blk, d), lambda i: (i, 0)),
            pl.BlockSpec((d, 64), lambda i: (0, 0)),
            pl.BlockSpec((1, 64), lambda i: (0, 0)),
            pl.BlockSpec((d, 64), lambda i: (0, 0)),
            pl.BlockSpec((d, 8), lambda i: (0, 0)),
            pl.BlockSpec((1, 8), lambda i: (0, 0)),
        ],
        out_specs=[
            pl.BlockSpec((blk, 64), lambda i: (i, 0)),
            pl.BlockSpec((blk, 64), lambda i: (i, 0)),
            pl.BlockSpec((blk, 8), lambda i: (i, 0)),
        ],
        out_shape=[
            jax.ShapeDtypeStruct((bb, 64), jnp.float32),
            jax.ShapeDtypeStruct((bb, 64), jnp.float32),
            jax.ShapeDtypeStruct((bb, 8), jnp.float32),
        ],
    )(x2, ws, bs, wd, wb, bb_)


def kernel(node_ids, rel_ids, center_mol_idx, non_molecule_node_ids, edge_index,
           node_emb_table, rel_emb_table, lin_W, lin_b,
           conv1_W, conv1_b, conv2_W, conv2_b,
           ec_W, ec_b, mp_W, mp_b, nc_W, nc_b, bp_W, bp_b):
    f32 = jnp.float32
    i32 = jnp.int32
    node_ids = node_ids.astype(i32)
    rel_ids = rel_ids.astype(i32)
    src = edge_index[0].astype(i32)
    dst = edge_index[1].astype(i32)

    epw = E // NW
    ppw = E_PAD // NW - epw
    pad_s = jnp.full((NW, ppw), N, i32)
    src_p = jnp.concatenate([src.reshape(NW, epw), pad_s], 1).reshape(-1)
    rel_p = jnp.concatenate([rel_ids.reshape(NW, epw),
                             jnp.zeros((NW, ppw), i32)], 1).reshape(-1)
    pad_d = ((jnp.arange(NW * ppw, dtype=i32) * 1337) % NP).reshape(NW, ppw)
    dst_p = jnp.concatenate([dst.reshape(NW, epw), pad_d], 1).reshape(-1)
    npad = E_PAD - E
    src_q = jnp.concatenate([src, jnp.zeros((npad,), i32)])
    dst_q = jnp.concatenate([dst, jnp.zeros((npad,), i32)])

    ids_pad = jnp.concatenate([node_ids, jnp.zeros((NP - N,), i32)])
    rows = _sc_gather_rows(node_emb_table.astype(f32), ids_pad, 64)

    lin_bt = lin_b.reshape(1, -1)
    x0 = _tc_dense(rows, None, lin_W.T, lin_bt, True, True, 512, mask_pad=True)
    ea_u = _tc_dense(rel_emb_table.astype(f32), None, lin_W.T, lin_bt, True, True, 64)

    zblk = jnp.zeros((CH, D), f32)
    agg1 = _sc_msgpass(x0, ea_u, src_p, dst_p, rel_p, zblk)
    x1 = _tc_dense(x0, agg1, conv1_W.T, conv1_b.reshape(1, -1), False, True, 512,
                   mask_pad=True)
    agg2 = _sc_msgpass(x1, ea_u, src_p, dst_p, rel_p, zblk)
    x2 = _tc_dense(x1, agg2, conv2_W.T, conv2_b.reshape(1, -1), False, False, 512)

    wb = jnp.pad(bp_W.T, ((0, 0), (0, 7)))
    bb_ = jnp.pad(bp_b, (0, 7)).reshape(1, -1)
    s_head, d_head, bpad = _tc_heads(
        x2, ec_W[:, :D].T, ec_b.reshape(1, -1), ec_W[:, D:].T, wb, bb_)
    binary_pred = bpad[:N, :1]

    edge_class = _sc_edge_head(s_head, d_head, src_q, dst_q)

    gidx = jnp.concatenate([center_mol_idx.astype(i32),
                            non_molecule_node_ids.astype(i32)])
    xg = _sc_gather_rows(x2, gidx, 96)
    mp_wt = jnp.pad(mp_W.T, ((0, 0), (0, 28)))
    mp_bp = jnp.pad(mp_b, (0, 28)).reshape(1, -1)
    motif_pred = _tc_dense(xg, None, mp_wt, mp_bp, False, False, 512)[:1024, :100]
    nc_wt = jnp.pad(nc_W.T, ((0, 0), (0, 113)))
    nc_bp = jnp.pad(nc_b, (0, 113)).reshape(1, -1)
    node_class = _tc_dense(xg, None, nc_wt, nc_bp, False, False, 512, row_off=2)[:, :15]

    return (edge_class, motif_pred, node_class, binary_pred)


# speedup vs baseline: 1.2369x; 1.2243x over previous
import functools

import jax
import jax.numpy as jnp
from jax import lax
from jax.experimental import pallas as pl
from jax.experimental.pallas import tpu as pltpu
from jax.experimental.pallas import tpu_sc as plsc

D = 128
N = 10000
NP = 10240
E = 320000
NC = 2
NS = 16
NW = NC * NS
CH = 128
NCHW = 80
E_PAD = NW * NCHW * CH
REAL_CROWS = E // CH
ROWS_PER_TILE = NP // NS


def _mesh():
    return plsc.VectorSubcoreMesh(core_axis_name="c", subcore_axis_name="s")


def _wid():
    return lax.axis_index("s") * NC + lax.axis_index("c")


def _sc_gather_rows(table, idx, ch):
    (b,) = idx.shape
    _, d = table.shape
    bpw = b // NW
    nch = bpw // ch

    @functools.partial(
        pl.kernel,
        out_type=jax.ShapeDtypeStruct((b, d), jnp.float32),
        mesh=_mesh(),
        scratch_types=[
            pltpu.VMEM((ch,), jnp.int32),
            pltpu.VMEM((ch, d), jnp.float32),
            pltpu.SemaphoreType.DMA,
        ],
    )
    def k(table_h, idx_h, out_h, idx_v, rows_v, sem):
        base = _wid() * bpw

        def body(j, carry):
            off = pl.multiple_of(base + j * ch, 8)
            pltpu.sync_copy(idx_h.at[pl.ds(off, ch)], idx_v)
            pltpu.async_copy(table_h.at[idx_v], rows_v, sem).wait()
            pltpu.sync_copy(rows_v, out_h.at[pl.ds(off, ch)])
            return carry

        lax.fori_loop(0, nch, body, 0)

    return k(table, idx)


def _sc_msgpass(x, ea, src, dst, rel, zblk):
    MCH = 64
    MNCH = E_PAD // (NW * MCH)
    EW = MNCH * MCH

    @functools.partial(
        pl.kernel,
        out_type=jax.ShapeDtypeStruct((NC, NP, D), jnp.float32),
        mesh=_mesh(),
        scratch_types=[
            pltpu.VMEM((MCH,), jnp.int32),
            pltpu.VMEM((MCH,), jnp.int32),
            pltpu.VMEM((MCH,), jnp.int32),
            pltpu.VMEM((MCH,), jnp.int32),
            pltpu.VMEM((MCH,), jnp.int32),
            pltpu.VMEM((MCH,), jnp.int32),
            pltpu.VMEM((MCH, D), jnp.float32),
            pltpu.VMEM((MCH, D), jnp.float32),
            pltpu.VMEM((MCH, D), jnp.float32),
            pltpu.VMEM((MCH, D), jnp.float32),
            pltpu.VMEM_SHARED((NP, D), jnp.float32),
            pltpu.VMEM_SHARED((64, D), jnp.float32),
            pltpu.SemaphoreType.DMA,
            pltpu.SemaphoreType.DMA,
            pltpu.SemaphoreType.DMA,
            pltpu.SemaphoreType.DMA,
            pltpu.SemaphoreType.DMA,
            pltpu.SemaphoreType.DMA,
        ],
    )
    def k(x_h, ea_h, src_h, dst_h, rel_h, z_h, out_h,
          sia, sib, ria, rib, dia, dib, xsa, xsb, eaa, eab, agg, ea_sh,
          sxa, sxb, sea, seb, sda, sdb):
        cid = lax.axis_index("c")
        sid = lax.axis_index("s")
        base = (sid * NC + cid) * EW

        @pl.when(sid == 0)
        def _():
            pltpu.sync_copy(ea_h, ea_sh)

        def zbody(t, carry):
            r0 = pl.multiple_of(sid * ROWS_PER_TILE + t * CH, 8)
            pltpu.sync_copy(z_h, agg.at[pl.ds(r0, CH)])
            return carry

        lax.fori_loop(0, ROWS_PER_TILE // CH, zbody, 0)
        plsc.subcore_barrier()

        def issue_idx(j, si_, ri_, di_, sd):
            off = pl.multiple_of(base + j * MCH, 8)
            pltpu.async_copy(src_h.at[pl.ds(off, MCH)], si_, sd)
            pltpu.async_copy(rel_h.at[pl.ds(off, MCH)], ri_, sd)
            pltpu.async_copy(dst_h.at[pl.ds(off, MCH)], di_, sd)

        def wait_idx(j, si_, ri_, di_, sd):
            off = pl.multiple_of(base + j * MCH, 8)
            pltpu.make_async_copy(src_h.at[pl.ds(off, MCH)], si_, sd).wait()
            pltpu.make_async_copy(rel_h.at[pl.ds(off, MCH)], ri_, sd).wait()
            pltpu.make_async_copy(dst_h.at[pl.ds(off, MCH)], di_, sd).wait()

        def issue_gath(si_, ri_, xs_, ea_, sx, se):
            pltpu.async_copy(x_h.at[si_], xs_, sx)
            pltpu.async_copy(ea_sh.at[ri_], ea_, se)

        def wait_gath(si_, ri_, xs_, ea_, sx, se):
            pltpu.make_async_copy(x_h.at[si_], xs_, sx).wait()
            pltpu.make_async_copy(ea_sh.at[ri_], ea_, se).wait()

        def compute_scatter(xs_, ea_, di_):
            def cbody(i, carry):
                for jj in range(D // 16):
                    sl = pl.ds(jj * 16, 16)
                    xs_[i, sl] = jnp.maximum(xs_[i, sl] + ea_[i, sl], 0.0)
                return carry

            lax.fori_loop(0, MCH, cbody, 0)
            pltpu.sync_copy(xs_, agg.at[di_], add=True)

        issue_idx(0, sia, ria, dia, sda)
        issue_idx(1, sib, rib, dib, sdb)
        wait_idx(0, sia, ria, dia, sda)
        issue_gath(sia, ria, xsa, eaa, sxa, sea)
        last_t = MNCH // 2 - 1

        def ebody(t, carry):
            c0 = 2 * t
            c1 = 2 * t + 1
            wait_idx(c1, sib, rib, dib, sdb)
            issue_gath(sib, rib, xsb, eab, sxb, seb)
            wait_gath(sia, ria, xsa, eaa, sxa, sea)
            compute_scatter(xsa, eaa, dia)

            @pl.when(t < last_t)
            def _():
                issue_idx(c0 + 2, sia, ria, dia, sda)

            wait_gath(sib, rib, xsb, eab, sxb, seb)

            @pl.when(t < last_t)
            def _():
                wait_idx(c0 + 2, sia, ria, dia, sda)
                issue_gath(sia, ria, xsa, eaa, sxa, sea)

            compute_scatter(xsb, eab, dib)

            @pl.when(t < last_t)
            def _():
                issue_idx(c1 + 2, sib, rib, dib, sdb)

            return carry

        lax.fori_loop(0, MNCH // 2, ebody, 0)

        plsc.subcore_barrier()

        def obody(t, carry):
            r0 = pl.multiple_of(sid * ROWS_PER_TILE + t * CH, 8)
            pltpu.sync_copy(agg.at[pl.ds(r0, CH)], out_h.at[cid, pl.ds(r0, CH)])
            return carry

        lax.fori_loop(0, ROWS_PER_TILE // CH, obody, 0)

    return k(x, ea, src, dst, rel, zblk)


def _sc_edge_head(s_head, d_head, src, dst):
    k64 = 64
    EW = NCHW * CH

    @functools.partial(
        pl.kernel,
        out_type=jax.ShapeDtypeStruct((E, k64), jnp.float32),
        mesh=_mesh(),
        scratch_types=[
            pltpu.VMEM((EW,), jnp.int32),
            pltpu.VMEM((EW,), jnp.int32),
            pltpu.VMEM((CH, k64), jnp.float32),
            pltpu.VMEM((CH, k64), jnp.float32),
            pltpu.VMEM((CH, k64), jnp.float32),
            pltpu.VMEM((CH, k64), jnp.float32),
            pltpu.SemaphoreType.DMA,
            pltpu.SemaphoreType.DMA,
            pltpu.SemaphoreType.DMA,
            pltpu.SemaphoreType.DMA,
            pltpu.SemaphoreType.DMA,
            pltpu.SemaphoreType.DMA,
        ],
        compiler_params=pltpu.CompilerParams(use_tc_tiling_on_sc=False),
    )
    def k(s_h, d_h, src_h, dst_h, out_h,
          si, di, sva, svb, dva, dvb, ssa, ssb, sda, sdb, swa, swb):
        crow0 = _wid() * NCHW
        base = crow0 * CH

        pltpu.sync_copy(src_h.at[pl.ds(base, EW)], si)
        pltpu.sync_copy(dst_h.at[pl.ds(base, EW)], di)

        def issue(j, sv_, dv_, ss, sd):
            iw = pl.multiple_of(j * CH, 8)
            pltpu.async_copy(s_h.at[si.at[pl.ds(iw, CH)]], sv_, ss)
            pltpu.async_copy(d_h.at[di.at[pl.ds(iw, CH)]], dv_, sd)

        def wait(j, sv_, dv_, ss, sd):
            iw = pl.multiple_of(j * CH, 8)
            pltpu.make_async_copy(s_h.at[si.at[pl.ds(iw, CH)]], sv_, ss).wait()
            pltpu.make_async_copy(d_h.at[di.at[pl.ds(iw, CH)]], dv_, sd).wait()

        def compute_store(j, sv_, dv_, sw):
            def cbody(i, carry):
                for jj in range(k64 // 16):
                    sl = pl.ds(jj * 16, 16)
                    sv_[i, sl] = sv_[i, sl] + dv_[i, sl]
                return carry

            lax.fori_loop(0, CH, cbody, 0)

            @pl.when(crow0 + j < REAL_CROWS)
            def _():
                off = pl.multiple_of((crow0 + j) * CH, 8)
                pltpu.async_copy(sv_, out_h.at[pl.ds(off, CH)], sw)

        def wait_store(j, sv_, sw):
            @pl.when(crow0 + j < REAL_CROWS)
            def _():
                off = pl.multiple_of((crow0 + j) * CH, 8)
                pltpu.make_async_copy(sv_, out_h.at[pl.ds(off, CH)], sw).wait()

        issue(0, sva, dva, ssa, sda)
        last_t = NCHW // 2 - 1

        def ebody(t, carry):
            c0 = 2 * t
            c1 = 2 * t + 1

            @pl.when(t > 0)
            def _():
                wait_store(c1 - 2, svb, swb)

            issue(c1, svb, dvb, ssb, sdb)
            wait(c0, sva, dva, ssa, sda)
            compute_store(c0, sva, dva, swa)

            @pl.when(t < last_t)
            def _():
                wait_store(c0, sva, swa)
                issue(c0 + 2, sva, dva, ssa, sda)

            wait(c1, svb, dvb, ssb, sdb)
            compute_store(c1, svb, dvb, swb)
            return carry

        lax.fori_loop(0, NCHW // 2, ebody, 0)
        wait_store(NCHW - 2, sva, swa)
        wait_store(NCHW - 1, svb, swb)

    return k(s_head, d_head, src, dst)


def _tc_dense(x, agg, wt, b, do_norm, do_relu, blk, row_off=0, mask_pad=False):
    bb, d = x.shape
    kk = wt.shape[1]
    has_agg = agg is not None
    nrows = bb - row_off * blk

    def body(*refs):
        xv = refs[0][...]
        if has_agg:
            av = refs[1][...]
            xv = xv + av[0] + av[1]
        iw = 1 + int(has_agg)
        if do_norm:
            s = jnp.sum(xv * xv, axis=1, keepdims=True)
            xv = xv / jnp.maximum(jnp.sqrt(s), 1e-12)
        y = jnp.dot(xv, refs[iw][...], preferred_element_type=jnp.float32)
        y = y + refs[iw + 1][...]
        if do_relu:
            y = jnp.maximum(y, 0.0)
        if mask_pad:
            rows = (jax.lax.broadcasted_iota(jnp.int32, (blk, kk), 0)
                    + pl.program_id(0) * blk)
            y = jnp.where(rows < N, y, -1e9)
        refs[iw + 2][...] = y

    in_specs = [pl.BlockSpec((blk, d), lambda i: (i + row_off, 0))]
    if has_agg:
        in_specs.append(pl.BlockSpec((NC, blk, d), lambda i: (0, i + row_off, 0)))
    in_specs += [
        pl.BlockSpec((d, kk), lambda i: (0, 0)),
        pl.BlockSpec((1, kk), lambda i: (0, 0)),
    ]
    args = (x, agg, wt, b) if has_agg else (x, wt, b)
    return pl.pallas_call(
        body,
        grid=(nrows // blk,),
        in_specs=in_specs,
        out_specs=pl.BlockSpec((blk, kk), lambda i: (i, 0)),
        out_shape=jax.ShapeDtypeStruct((nrows, kk), jnp.float32),
    )(*args)


def _tc_heads(x2, ws, bs, wd, wb, bb_):
    bb, d = x2.shape
    blk = 512

    def body(x_ref, ws_ref, bs_ref, wd_ref, wb_ref, bb_ref, s_ref, d_ref, b_ref):
        xv = x_ref[...]
        s_ref[...] = jnp.dot(xv, ws_ref[...], preferred_element_type=jnp.float32) + bs_ref[...]
        d_ref[...] = jnp.dot(xv, wd_ref[...], preferred_element_type=jnp.float32)
        b_ref[...] = jnp.dot(xv, wb_ref[...], preferred_element_type=jnp.float32) + bb_ref[...]

    return pl.pallas_call(
        body,
        grid=(bb // blk,),
        in_specs=[
            pl.BlockSpec((blk, d), lambda i: (i, 0)),
            pl.BlockSpec((d, 64), lambda i: (0, 0)),
            pl.BlockSpec((1, 64), lambda i: (0, 0)),
            pl.BlockSpec((d, 64), lambda i: (0, 0)),
            pl.BlockSpec((d, 8), lambda i: (0, 0)),
            pl.BlockSpec((1, 8), lambda i: (0, 0)),
        ],
        out_specs=[
            pl.BlockSpec((blk, 64), lambda i: (i, 0)),
            pl.BlockSpec((blk, 64), lambda i: (i, 0)),
            pl.BlockSpec((blk, 8), lambda i: (i, 0)),
        ],
        out_shape=[
            jax.ShapeDtypeStruct((bb, 64), jnp.float32),
            jax.ShapeDtypeStruct((bb, 64), jnp.float32),
            jax.ShapeDtypeStruct((bb, 8), jnp.float32),
        ],
    )(x2, ws, bs, wd, wb, bb_)


def kernel(node_ids, rel_ids, center_mol_idx, non_molecule_node_ids, edge_index,
           node_emb_table, rel_emb_table, lin_W, lin_b,
           conv1_W, conv1_b, conv2_W, conv2_b,
           ec_W, ec_b, mp_W, mp_b, nc_W, nc_b, bp_W, bp_b):
    f32 = jnp.float32
    i32 = jnp.int32
    node_ids = node_ids.astype(i32)
    rel_ids = rel_ids.astype(i32)
    src = edge_index[0].astype(i32)
    dst = edge_index[1].astype(i32)

    epw = E // NW
    ppw = E_PAD // NW - epw
    pad_s = jnp.full((NW, ppw), N, i32)
    src_p = jnp.concatenate([src.reshape(NW, epw), pad_s], 1).reshape(-1)
    rel_p = jnp.concatenate([rel_ids.reshape(NW, epw),
                             jnp.zeros((NW, ppw), i32)], 1).reshape(-1)
    pad_d = ((jnp.arange(NW * ppw, dtype=i32) * 1337) % NP).reshape(NW, ppw)
    dst_p = jnp.concatenate([dst.reshape(NW, epw), pad_d], 1).reshape(-1)
    npad = E_PAD - E
    src_q = jnp.concatenate([src, jnp.zeros((npad,), i32)])
    dst_q = jnp.concatenate([dst, jnp.zeros((npad,), i32)])

    ids_pad = jnp.concatenate([node_ids, jnp.zeros((NP - N,), i32)])
    rows = _sc_gather_rows(node_emb_table.astype(f32), ids_pad, 64)

    lin_bt = lin_b.reshape(1, -1)
    x0 = _tc_dense(rows, None, lin_W.T, lin_bt, True, True, 512, mask_pad=True)
    ea_u = _tc_dense(rel_emb_table.astype(f32), None, lin_W.T, lin_bt, True, True, 64)

    zblk = jnp.zeros((CH, D), f32)
    agg1 = _sc_msgpass(x0, ea_u, src_p, dst_p, rel_p, zblk)
    x1 = _tc_dense(x0, agg1, conv1_W.T, conv1_b.reshape(1, -1), False, True, 512,
                   mask_pad=True)
    agg2 = _sc_msgpass(x1, ea_u, src_p, dst_p, rel_p, zblk)
    x2 = _tc_dense(x1, agg2, conv2_W.T, conv2_b.reshape(1, -1), False, False, 512)

    wb = jnp.pad(bp_W.T, ((0, 0), (0, 7)))
    bb_ = jnp.pad(bp_b, (0, 7)).reshape(1, -1)
    s_head, d_head, bpad = _tc_heads(
        x2, ec_W[:, :D].T, ec_b.reshape(1, -1), ec_W[:, D:].T, wb, bb_)
    binary_pred = bpad[:N, :1]

    edge_class = _sc_edge_head(s_head, d_head, src_q, dst_q)

    gidx = jnp.concatenate([center_mol_idx.astype(i32),
                            non_molecule_node_ids.astype(i32)])
    xg = _sc_gather_rows(x2, gidx, 96)
    mp_wt = jnp.pad(mp_W.T, ((0, 0), (0, 28)))
    mp_bp = jnp.pad(mp_b, (0, 28)).reshape(1, -1)
    motif_pred = _tc_dense(xg, None, mp_wt, mp_bp, False, False, 512)[:1024, :100]
    nc_wt = jnp.pad(nc_W.T, ((0, 0), (0, 113)))
    nc_bp = jnp.pad(nc_b, (0, 113)).reshape(1, -1)
    node_class = _tc_dense(xg, None, nc_wt, nc_bp, False, False, 512, row_off=2)[:, :15]

    return (edge_class, motif_pred, node_class, binary_pred)


# speedup vs baseline: 1.4201x; 1.1482x over previous
import functools

import jax
import jax.numpy as jnp
from jax import lax
from jax.experimental import pallas as pl
from jax.experimental.pallas import tpu as pltpu
from jax.experimental.pallas import tpu_sc as plsc

D = 128
N = 10000
NP = 10240
E = 320000
NC = 2
NS = 16
NW = NC * NS
CH = 128
NCHW = 80
E_PAD = NW * NCHW * CH
REAL_CROWS = E // CH
ROWS_PER_TILE = NP // NS


def _mesh():
    return plsc.VectorSubcoreMesh(core_axis_name="c", subcore_axis_name="s")


def _wid():
    return lax.axis_index("s") * NC + lax.axis_index("c")


def _sc_gather_rows(table, idx, ch):
    (b,) = idx.shape
    _, d = table.shape
    bpw = b // NW
    nch = bpw // ch

    @functools.partial(
        pl.kernel,
        out_type=jax.ShapeDtypeStruct((b, d), jnp.float32),
        mesh=_mesh(),
        scratch_types=[
            pltpu.VMEM((ch,), jnp.int32),
            pltpu.VMEM((ch, d), jnp.float32),
            pltpu.SemaphoreType.DMA,
        ],
    )
    def k(table_h, idx_h, out_h, idx_v, rows_v, sem):
        base = _wid() * bpw

        def body(j, carry):
            off = pl.multiple_of(base + j * ch, 8)
            pltpu.sync_copy(idx_h.at[pl.ds(off, ch)], idx_v)
            pltpu.async_copy(table_h.at[idx_v], rows_v, sem).wait()
            pltpu.sync_copy(rows_v, out_h.at[pl.ds(off, ch)])
            return carry

        lax.fori_loop(0, nch, body, 0)

    return k(table, idx)


def _sc_msgpass(x, ea, src, dst, rel, zblk):
    MCH = 64
    MNCH = E_PAD // (NW * MCH)
    EW = MNCH * MCH

    @functools.partial(
        pl.kernel,
        out_type=jax.ShapeDtypeStruct((NC, NP, D), jnp.float32),
        mesh=_mesh(),
        scratch_types=[
            pltpu.VMEM((MCH,), jnp.int32),
            pltpu.VMEM((MCH,), jnp.int32),
            pltpu.VMEM((MCH,), jnp.int32),
            pltpu.VMEM((MCH,), jnp.int32),
            pltpu.VMEM((MCH,), jnp.int32),
            pltpu.VMEM((MCH,), jnp.int32),
            pltpu.VMEM((MCH, D), jnp.float32),
            pltpu.VMEM((MCH, D), jnp.float32),
            pltpu.VMEM((MCH, D), jnp.float32),
            pltpu.VMEM((MCH, D), jnp.float32),
            pltpu.VMEM_SHARED((NP, D), jnp.float32),
            pltpu.VMEM_SHARED((64, D), jnp.float32),
            pltpu.SemaphoreType.DMA,
            pltpu.SemaphoreType.DMA,
            pltpu.SemaphoreType.DMA,
            pltpu.SemaphoreType.DMA,
            pltpu.SemaphoreType.DMA,
            pltpu.SemaphoreType.DMA,
        ],
    )
    def k(x_h, ea_h, src_h, dst_h, rel_h, z_h, out_h,
          sia, sib, ria, rib, dia, dib, xsa, xsb, eaa, eab, agg, ea_sh,
          sxa, sxb, sea, seb, sda, sdb):
        cid = lax.axis_index("c")
        sid = lax.axis_index("s")
        base = (sid * NC + cid) * EW

        @pl.when(sid == 0)
        def _():
            pltpu.sync_copy(ea_h, ea_sh)

        def zbody(t, carry):
            r0 = pl.multiple_of(sid * ROWS_PER_TILE + t * CH, 8)
            pltpu.sync_copy(z_h, agg.at[pl.ds(r0, CH)])
            return carry

        lax.fori_loop(0, ROWS_PER_TILE // CH, zbody, 0)
        plsc.subcore_barrier()

        def issue_idx(j, si_, ri_, di_, sd):
            off = pl.multiple_of(base + j * MCH, 8)
            pltpu.async_copy(src_h.at[pl.ds(off, MCH)], si_, sd)
            pltpu.async_copy(rel_h.at[pl.ds(off, MCH)], ri_, sd)
            pltpu.async_copy(dst_h.at[pl.ds(off, MCH)], di_, sd)

        def wait_idx(j, si_, ri_, di_, sd):
            off = pl.multiple_of(base + j * MCH, 8)
            pltpu.make_async_copy(src_h.at[pl.ds(off, MCH)], si_, sd).wait()
            pltpu.make_async_copy(rel_h.at[pl.ds(off, MCH)], ri_, sd).wait()
            pltpu.make_async_copy(dst_h.at[pl.ds(off, MCH)], di_, sd).wait()

        def issue_gath(si_, ri_, xs_, ea_, sx, se):
            pltpu.async_copy(x_h.at[si_], xs_, sx)
            pltpu.async_copy(ea_sh.at[ri_], ea_, se)

        def wait_gath(si_, ri_, xs_, ea_, sx, se):
            pltpu.make_async_copy(x_h.at[si_], xs_, sx).wait()
            pltpu.make_async_copy(ea_sh.at[ri_], ea_, se).wait()

        def compute_scatter(xs_, ea_, di_):
            def cbody(i, carry):
                for jj in range(D // 16):
                    sl = pl.ds(jj * 16, 16)
                    xs_[i, sl] = jnp.maximum(xs_[i, sl] + ea_[i, sl], 0.0)
                return carry

            lax.fori_loop(0, MCH, cbody, 0)
            pltpu.sync_copy(xs_, agg.at[di_], add=True)

        issue_idx(0, sia, ria, dia, sda)
        issue_idx(1, sib, rib, dib, sdb)
        wait_idx(0, sia, ria, dia, sda)
        issue_gath(sia, ria, xsa, eaa, sxa, sea)
        last_t = MNCH // 2 - 1

        def ebody(t, carry):
            c0 = 2 * t
            c1 = 2 * t + 1
            wait_idx(c1, sib, rib, dib, sdb)
            issue_gath(sib, rib, xsb, eab, sxb, seb)
            wait_gath(sia, ria, xsa, eaa, sxa, sea)
            compute_scatter(xsa, eaa, dia)

            @pl.when(t < last_t)
            def _():
                issue_idx(c0 + 2, sia, ria, dia, sda)

            wait_gath(sib, rib, xsb, eab, sxb, seb)

            @pl.when(t < last_t)
            def _():
                wait_idx(c0 + 2, sia, ria, dia, sda)
                issue_gath(sia, ria, xsa, eaa, sxa, sea)

            compute_scatter(xsb, eab, dib)

            @pl.when(t < last_t)
            def _():
                issue_idx(c1 + 2, sib, rib, dib, sdb)

            return carry

        lax.fori_loop(0, MNCH // 2, ebody, 0)

        plsc.subcore_barrier()

        def obody(t, carry):
            r0 = pl.multiple_of(sid * ROWS_PER_TILE + t * CH, 8)
            pltpu.sync_copy(agg.at[pl.ds(r0, CH)], out_h.at[cid, pl.ds(r0, CH)])
            return carry

        lax.fori_loop(0, ROWS_PER_TILE // CH, obody, 0)

    return k(x, ea, src, dst, rel, zblk)


def _sc_edge_head(s_head, d_head, src, dst):
    k64 = 64
    EW = NCHW * CH
    ECH = 64
    ENCH = EW // ECH
    RC = E // ECH

    @functools.partial(
        pl.kernel,
        out_type=jax.ShapeDtypeStruct((E, k64), jnp.float32),
        mesh=_mesh(),
        scratch_types=[
            pltpu.VMEM((EW,), jnp.int32),
            pltpu.VMEM((EW,), jnp.int32),
            pltpu.VMEM((ECH, k64), jnp.float32),
            pltpu.VMEM((ECH, k64), jnp.float32),
            pltpu.VMEM((ECH, k64), jnp.float32),
            pltpu.VMEM((ECH, k64), jnp.float32),
            pltpu.VMEM_SHARED((NP, k64), jnp.float32),
            pltpu.VMEM_SHARED((NP, k64), jnp.float32),
            pltpu.SemaphoreType.DMA,
            pltpu.SemaphoreType.DMA,
            pltpu.SemaphoreType.DMA,
            pltpu.SemaphoreType.DMA,
            pltpu.SemaphoreType.DMA,
            pltpu.SemaphoreType.DMA,
        ],
        compiler_params=pltpu.CompilerParams(use_tc_tiling_on_sc=False),
    )
    def k(s_h, d_h, src_h, dst_h, out_h,
          si, di, sva, svb, dva, dvb, s_sh, d_sh, ssa, ssb, sda, sdb, swa, swb):
        sid = lax.axis_index("s")
        crow0 = _wid() * ENCH
        base = crow0 * ECH

        r0 = pl.multiple_of(sid * ROWS_PER_TILE, 8)
        pltpu.sync_copy(s_h.at[pl.ds(r0, ROWS_PER_TILE)],
                        s_sh.at[pl.ds(r0, ROWS_PER_TILE)])
        pltpu.sync_copy(d_h.at[pl.ds(r0, ROWS_PER_TILE)],
                        d_sh.at[pl.ds(r0, ROWS_PER_TILE)])
        pltpu.sync_copy(src_h.at[pl.ds(base, EW)], si)
        pltpu.sync_copy(dst_h.at[pl.ds(base, EW)], di)
        plsc.subcore_barrier()

        def issue(j, sv_, dv_, ss, sd):
            iw = pl.multiple_of(j * ECH, 8)
            pltpu.async_copy(s_sh.at[si.at[pl.ds(iw, ECH)]], sv_, ss)
            pltpu.async_copy(d_sh.at[di.at[pl.ds(iw, ECH)]], dv_, sd)

        def wait(j, sv_, dv_, ss, sd):
            iw = pl.multiple_of(j * ECH, 8)
            pltpu.make_async_copy(s_sh.at[si.at[pl.ds(iw, ECH)]], sv_, ss).wait()
            pltpu.make_async_copy(d_sh.at[di.at[pl.ds(iw, ECH)]], dv_, sd).wait()

        def compute_store(j, sv_, dv_, sw):
            def cbody(i, carry):
                for jj in range(k64 // 16):
                    sl = pl.ds(jj * 16, 16)
                    sv_[i, sl] = sv_[i, sl] + dv_[i, sl]
                return carry

            lax.fori_loop(0, ECH, cbody, 0)

            @pl.when(crow0 + j < RC)
            def _():
                off = pl.multiple_of((crow0 + j) * ECH, 8)
                pltpu.async_copy(sv_, out_h.at[pl.ds(off, ECH)], sw)

        def wait_store(j, sv_, sw):
            @pl.when(crow0 + j < RC)
            def _():
                off = pl.multiple_of((crow0 + j) * ECH, 8)
                pltpu.make_async_copy(sv_, out_h.at[pl.ds(off, ECH)], sw).wait()

        issue(0, sva, dva, ssa, sda)
        last_t = ENCH // 2 - 1

        def ebody(t, carry):
            c0 = 2 * t
            c1 = 2 * t + 1

            @pl.when(t > 0)
            def _():
                wait_store(c1 - 2, svb, swb)

            issue(c1, svb, dvb, ssb, sdb)
            wait(c0, sva, dva, ssa, sda)
            compute_store(c0, sva, dva, swa)

            @pl.when(t < last_t)
            def _():
                wait_store(c0, sva, swa)
                issue(c0 + 2, sva, dva, ssa, sda)

            wait(c1, svb, dvb, ssb, sdb)
            compute_store(c1, svb, dvb, swb)
            return carry

        lax.fori_loop(0, ENCH // 2, ebody, 0)
        wait_store(ENCH - 2, sva, swa)
        wait_store(ENCH - 1, svb, swb)

    return k(s_head, d_head, src, dst)


def _tc_dense(x, agg, wt, b, do_norm, do_relu, blk, row_off=0, mask_pad=False):
    bb, d = x.shape
    kk = wt.shape[1]
    has_agg = agg is not None
    nrows = bb - row_off * blk

    def body(*refs):
        xv = refs[0][...]
        if has_agg:
            av = refs[1][...]
            xv = xv + av[0] + av[1]
        iw = 1 + int(has_agg)
        if do_norm:
            s = jnp.sum(xv * xv, axis=1, keepdims=True)
            xv = xv / jnp.maximum(jnp.sqrt(s), 1e-12)
        y = jnp.dot(xv, refs[iw][...], preferred_element_type=jnp.float32)
        y = y + refs[iw + 1][...]
        if do_relu:
            y = jnp.maximum(y, 0.0)
        if mask_pad:
            rows = (jax.lax.broadcasted_iota(jnp.int32, (blk, kk), 0)
                    + pl.program_id(0) * blk)
            y = jnp.where(rows < N, y, -1e9)
        refs[iw + 2][...] = y

    in_specs = [pl.BlockSpec((blk, d), lambda i: (i + row_off, 0))]
    if has_agg:
        in_specs.append(pl.BlockSpec((NC, blk, d), lambda i: (0, i + row_off, 0)))
    in_specs += [
        pl.BlockSpec((d, kk), lambda i: (0, 0)),
        pl.BlockSpec((1, kk), lambda i: (0, 0)),
    ]
    args = (x, agg, wt, b) if has_agg else (x, wt, b)
    return pl.pallas_call(
        body,
        grid=(nrows // blk,),
        in_specs=in_specs,
        out_specs=pl.BlockSpec((blk, kk), lambda i: (i, 0)),
        out_shape=jax.ShapeDtypeStruct((nrows, kk), jnp.float32),
    )(*args)


def _tc_heads(x2, ws, bs, wd, wb, bb_):
    bb, d = x2.shape
    blk = 512

    def body(x_ref, ws_ref, bs_ref, wd_ref, wb_ref, bb_ref, s_ref, d_ref, b_ref):
        xv = x_ref[...]
        s_ref[...] = jnp.dot(xv, ws_ref[...], preferred_element_type=jnp.float32) + bs_ref[...]
        d_ref[...] = jnp.dot(xv, wd_ref[...], preferred_element_type=jnp.float32)
        b_ref[...] = jnp.dot(xv, wb_ref[...], preferred_element_type=jnp.float32) + bb_ref[...]

    return pl.pallas_call(
        body,
        grid=(bb // blk,),
        in_specs=[
            pl.BlockSpec((blk, d), lambda i: (i, 0)),
            pl.BlockSpec((d, 64), lambda i: (0, 0)),
            pl.BlockSpec((1, 64), lambda i: (0, 0)),
            pl.BlockSpec((d, 64), lambda i: (0, 0)),
            pl.BlockSpec((d, 8), lambda i: (0, 0)),
            pl.BlockSpec((1, 8), lambda i: (0, 0)),
        ],
        out_specs=[
            pl.BlockSpec((blk, 64), lambda i: (i, 0)),
            pl.BlockSpec((blk, 64), lambda i: (i, 0)),
            pl.BlockSpec((blk, 8), lambda i: (i, 0)),
        ],
        out_shape=[
            jax.ShapeDtypeStruct((bb, 64), jnp.float32),
            jax.ShapeDtypeStruct((bb, 64), jnp.float32),
            jax.ShapeDtypeStruct((bb, 8), jnp.float32),
        ],
    )(x2, ws, bs, wd, wb, bb_)


def kernel(node_ids, rel_ids, center_mol_idx, non_molecule_node_ids, edge_index,
           node_emb_table, rel_emb_table, lin_W, lin_b,
           conv1_W, conv1_b, conv2_W, conv2_b,
           ec_W, ec_b, mp_W, mp_b, nc_W, nc_b, bp_W, bp_b):
    f32 = jnp.float32
    i32 = jnp.int32
    node_ids = node_ids.astype(i32)
    rel_ids = rel_ids.astype(i32)
    src = edge_index[0].astype(i32)
    dst = edge_index[1].astype(i32)

    epw = E // NW
    ppw = E_PAD // NW - epw
    pad_s = jnp.full((NW, ppw), N, i32)
    src_p = jnp.concatenate([src.reshape(NW, epw), pad_s], 1).reshape(-1)
    rel_p = jnp.concatenate([rel_ids.reshape(NW, epw),
                             jnp.zeros((NW, ppw), i32)], 1).reshape(-1)
    pad_d = ((jnp.arange(NW * ppw, dtype=i32) * 1337) % NP).reshape(NW, ppw)
    dst_p = jnp.concatenate([dst.reshape(NW, epw), pad_d], 1).reshape(-1)
    npad = E_PAD - E
    src_q = jnp.concatenate([src, jnp.zeros((npad,), i32)])
    dst_q = jnp.concatenate([dst, jnp.zeros((npad,), i32)])

    ids_pad = jnp.concatenate([node_ids, jnp.zeros((NP - N,), i32)])
    rows = _sc_gather_rows(node_emb_table.astype(f32), ids_pad, 64)

    lin_bt = lin_b.reshape(1, -1)
    x0 = _tc_dense(rows, None, lin_W.T, lin_bt, True, True, 512, mask_pad=True)
    ea_u = _tc_dense(rel_emb_table.astype(f32), None, lin_W.T, lin_bt, True, True, 64)

    zblk = jnp.zeros((CH, D), f32)
    agg1 = _sc_msgpass(x0, ea_u, src_p, dst_p, rel_p, zblk)
    x1 = _tc_dense(x0, agg1, conv1_W.T, conv1_b.reshape(1, -1), False, True, 512,
                   mask_pad=True)
    agg2 = _sc_msgpass(x1, ea_u, src_p, dst_p, rel_p, zblk)
    x2 = _tc_dense(x1, agg2, conv2_W.T, conv2_b.reshape(1, -1), False, False, 512)

    wb = jnp.pad(bp_W.T, ((0, 0), (0, 7)))
    bb_ = jnp.pad(bp_b, (0, 7)).reshape(1, -1)
    s_head, d_head, bpad = _tc_heads(
        x2, ec_W[:, :D].T, ec_b.reshape(1, -1), ec_W[:, D:].T, wb, bb_)
    binary_pred = bpad[:N, :1]

    edge_class = _sc_edge_head(s_head, d_head, src_q, dst_q)

    gidx = jnp.concatenate([center_mol_idx.astype(i32),
                            non_molecule_node_ids.astype(i32)])
    xg = _sc_gather_rows(x2, gidx, 96)
    mp_wt = jnp.pad(mp_W.T, ((0, 0), (0, 28)))
    mp_bp = jnp.pad(mp_b, (0, 28)).reshape(1, -1)
    motif_pred = _tc_dense(xg, None, mp_wt, mp_bp, False, False, 512)[:1024, :100]
    nc_wt = jnp.pad(nc_W.T, ((0, 0), (0, 113)))
    nc_bp = jnp.pad(nc_b, (0, 113)).reshape(1, -1)
    node_class = _tc_dense(xg, None, nc_wt, nc_bp, False, False, 512, row_off=2)[:, :15]

    return (edge_class, motif_pred, node_class, binary_pred)


# speedup vs baseline: 1.4696x; 1.0349x over previous
import functools

import jax
import jax.numpy as jnp
from jax import lax
from jax.experimental import pallas as pl
from jax.experimental.pallas import tpu as pltpu
from jax.experimental.pallas import tpu_sc as plsc

D = 128
N = 10000
NP = 10240
E = 320000
NC = 2
NS = 16
NW = NC * NS
CH = 128
NCHW = 80
E_PAD = NW * NCHW * CH
REAL_CROWS = E // CH
ROWS_PER_TILE = NP // NS


def _mesh():
    return plsc.VectorSubcoreMesh(core_axis_name="c", subcore_axis_name="s")


def _wid():
    return lax.axis_index("s") * NC + lax.axis_index("c")


def _sc_gather_rows(table, idx, ch):
    (b,) = idx.shape
    _, d = table.shape
    bpw = b // NW
    nch = bpw // ch

    @functools.partial(
        pl.kernel,
        out_type=jax.ShapeDtypeStruct((b, d), jnp.float32),
        mesh=_mesh(),
        scratch_types=[
            pltpu.VMEM((ch,), jnp.int32),
            pltpu.VMEM((ch, d), jnp.float32),
            pltpu.SemaphoreType.DMA,
        ],
    )
    def k(table_h, idx_h, out_h, idx_v, rows_v, sem):
        base = _wid() * bpw

        def body(j, carry):
            off = pl.multiple_of(base + j * ch, 8)
            pltpu.sync_copy(idx_h.at[pl.ds(off, ch)], idx_v)
            pltpu.async_copy(table_h.at[idx_v], rows_v, sem).wait()
            pltpu.sync_copy(rows_v, out_h.at[pl.ds(off, ch)])
            return carry

        lax.fori_loop(0, nch, body, 0)

    return k(table, idx)


def _sc_msgpass(x, ea, src, dst, rel, zblk):
    MCH = 64
    MNCH = E_PAD // (NW * MCH)
    EW = MNCH * MCH

    @functools.partial(
        pl.kernel,
        out_type=jax.ShapeDtypeStruct((NC, NP, D), jnp.float32),
        mesh=_mesh(),
        scratch_types=[
            pltpu.VMEM((MCH,), jnp.int32),
            pltpu.VMEM((MCH,), jnp.int32),
            pltpu.VMEM((MCH,), jnp.int32),
            pltpu.VMEM((MCH,), jnp.int32),
            pltpu.VMEM((MCH,), jnp.int32),
            pltpu.VMEM((MCH,), jnp.int32),
            pltpu.VMEM((MCH, D), jnp.float32),
            pltpu.VMEM((MCH, D), jnp.float32),
            pltpu.VMEM((MCH, D), jnp.float32),
            pltpu.VMEM((MCH, D), jnp.float32),
            pltpu.VMEM((MCH,), jnp.int32),
            pltpu.VMEM((MCH,), jnp.int32),
            pltpu.VMEM_SHARED((NP, D), jnp.float32),
            pltpu.VMEM_SHARED((64, D), jnp.float32),
            pltpu.SemaphoreType.DMA,
            pltpu.SemaphoreType.DMA,
            pltpu.SemaphoreType.DMA,
            pltpu.SemaphoreType.DMA,
            pltpu.SemaphoreType.DMA,
            pltpu.SemaphoreType.DMA,
            pltpu.SemaphoreType.DMA,
            pltpu.SemaphoreType.DMA,
        ],
    )
    def k(x_h, ea_h, src_h, dst_h, rel_h, z_h, out_h,
          sia, sib, ria, rib, dia, dib, xsa, xsb, eaa, eab, dsa, dsb, agg, ea_sh,
          sxa, sxb, sea, seb, sda, sdb, ssca, sscb):
        cid = lax.axis_index("c")
        sid = lax.axis_index("s")
        base = (sid * NC + cid) * EW

        @pl.when(sid == 0)
        def _():
            pltpu.sync_copy(ea_h, ea_sh)

        def zbody(t, carry):
            r0 = pl.multiple_of(sid * ROWS_PER_TILE + t * CH, 8)
            pltpu.sync_copy(z_h, agg.at[pl.ds(r0, CH)])
            return carry

        lax.fori_loop(0, ROWS_PER_TILE // CH, zbody, 0)
        plsc.subcore_barrier()

        def issue_idx(j, si_, ri_, di_, sd):
            off = pl.multiple_of(base + j * MCH, 8)
            pltpu.async_copy(src_h.at[pl.ds(off, MCH)], si_, sd)
            pltpu.async_copy(rel_h.at[pl.ds(off, MCH)], ri_, sd)
            pltpu.async_copy(dst_h.at[pl.ds(off, MCH)], di_, sd)

        def wait_idx(j, si_, ri_, di_, sd):
            off = pl.multiple_of(base + j * MCH, 8)
            pltpu.make_async_copy(src_h.at[pl.ds(off, MCH)], si_, sd).wait()
            pltpu.make_async_copy(rel_h.at[pl.ds(off, MCH)], ri_, sd).wait()
            pltpu.make_async_copy(dst_h.at[pl.ds(off, MCH)], di_, sd).wait()

        def issue_gath(si_, ri_, xs_, ea_, sx, se):
            pltpu.async_copy(x_h.at[si_], xs_, sx)
            pltpu.async_copy(ea_sh.at[ri_], ea_, se)

        def wait_gath(si_, ri_, xs_, ea_, sx, se):
            pltpu.make_async_copy(x_h.at[si_], xs_, sx).wait()
            pltpu.make_async_copy(ea_sh.at[ri_], ea_, se).wait()

        def compute(xs_, ea_, di_, ds_):
            def cbody(i, carry):
                for jj in range(D // 16):
                    sl = pl.ds(jj * 16, 16)
                    ea_[i, sl] = jnp.maximum(xs_[i, sl] + ea_[i, sl], 0.0)
                return carry

            lax.fori_loop(0, MCH, cbody, 0)
            for jj in range(MCH // 16):
                sl = pl.ds(jj * 16, 16)
                ds_[sl] = di_[sl]

        def scat(ea_, ds_, ssc):
            pltpu.async_copy(ea_, agg.at[ds_], ssc, add=True)

        def wait_scat(ea_, ds_, ssc):
            pltpu.make_async_copy(ea_, agg.at[ds_], ssc).wait()

        issue_idx(0, sia, ria, dia, sda)
        issue_idx(1, sib, rib, dib, sdb)
        wait_idx(0, sia, ria, dia, sda)
        issue_gath(sia, ria, xsa, eaa, sxa, sea)
        last_t = MNCH // 2 - 1

        def ebody(t, carry):
            c0 = 2 * t
            c1 = 2 * t + 1
            wait_idx(c1, sib, rib, dib, sdb)

            @pl.when(t > 0)
            def _():
                wait_scat(eab, dsb, sscb)

            issue_gath(sib, rib, xsb, eab, sxb, seb)
            wait_gath(sia, ria, xsa, eaa, sxa, sea)
            compute(xsa, eaa, dia, dsa)
            scat(eaa, dsa, ssca)

            @pl.when(t < last_t)
            def _():
                issue_idx(c0 + 2, sia, ria, dia, sda)

            wait_gath(sib, rib, xsb, eab, sxb, seb)

            @pl.when(t < last_t)
            def _():
                wait_idx(c0 + 2, sia, ria, dia, sda)
                wait_scat(eaa, dsa, ssca)
                issue_gath(sia, ria, xsa, eaa, sxa, sea)

            compute(xsb, eab, dib, dsb)
            scat(eab, dsb, sscb)

            @pl.when(t < last_t)
            def _():
                issue_idx(c1 + 2, sib, rib, dib, sdb)

            return carry

        lax.fori_loop(0, MNCH // 2, ebody, 0)
        wait_scat(eaa, dsa, ssca)
        wait_scat(eab, dsb, sscb)

        plsc.subcore_barrier()

        def obody(t, carry):
            r0 = pl.multiple_of(sid * ROWS_PER_TILE + t * CH, 8)
            pltpu.sync_copy(agg.at[pl.ds(r0, CH)], out_h.at[cid, pl.ds(r0, CH)])
            return carry

        lax.fori_loop(0, ROWS_PER_TILE // CH, obody, 0)

    return k(x, ea, src, dst, rel, zblk)


def _sc_edge_head(s_head, d_head, src, dst):
    k64 = 64
    EW = NCHW * CH
    ECH = 64
    ENCH = EW // ECH
    RC = E // ECH

    @functools.partial(
        pl.kernel,
        out_type=jax.ShapeDtypeStruct((E, k64), jnp.float32),
        mesh=_mesh(),
        scratch_types=[
            pltpu.VMEM((EW,), jnp.int32),
            pltpu.VMEM((EW,), jnp.int32),
            pltpu.VMEM((ECH, k64), jnp.float32),
            pltpu.VMEM((ECH, k64), jnp.float32),
            pltpu.VMEM((ECH, k64), jnp.float32),
            pltpu.VMEM((ECH, k64), jnp.float32),
            pltpu.VMEM_SHARED((NP, k64), jnp.float32),
            pltpu.VMEM_SHARED((NP, k64), jnp.float32),
            pltpu.SemaphoreType.DMA,
            pltpu.SemaphoreType.DMA,
            pltpu.SemaphoreType.DMA,
            pltpu.SemaphoreType.DMA,
            pltpu.SemaphoreType.DMA,
            pltpu.SemaphoreType.DMA,
        ],
        compiler_params=pltpu.CompilerParams(use_tc_tiling_on_sc=False),
    )
    def k(s_h, d_h, src_h, dst_h, out_h,
          si, di, sva, svb, dva, dvb, s_sh, d_sh, ssa, ssb, sda, sdb, swa, swb):
        sid = lax.axis_index("s")
        crow0 = _wid() * ENCH
        base = crow0 * ECH

        r0 = pl.multiple_of(sid * ROWS_PER_TILE, 8)
        pltpu.sync_copy(s_h.at[pl.ds(r0, ROWS_PER_TILE)],
                        s_sh.at[pl.ds(r0, ROWS_PER_TILE)])
        pltpu.sync_copy(d_h.at[pl.ds(r0, ROWS_PER_TILE)],
                        d_sh.at[pl.ds(r0, ROWS_PER_TILE)])
        pltpu.sync_copy(src_h.at[pl.ds(base, EW)], si)
        pltpu.sync_copy(dst_h.at[pl.ds(base, EW)], di)
        plsc.subcore_barrier()

        def issue(j, sv_, dv_, ss, sd):
            iw = pl.multiple_of(j * ECH, 8)
            pltpu.async_copy(s_sh.at[si.at[pl.ds(iw, ECH)]], sv_, ss)
            pltpu.async_copy(d_sh.at[di.at[pl.ds(iw, ECH)]], dv_, sd)

        def wait(j, sv_, dv_, ss, sd):
            iw = pl.multiple_of(j * ECH, 8)
            pltpu.make_async_copy(s_sh.at[si.at[pl.ds(iw, ECH)]], sv_, ss).wait()
            pltpu.make_async_copy(d_sh.at[di.at[pl.ds(iw, ECH)]], dv_, sd).wait()

        def compute_store(j, sv_, dv_, sw):
            def cbody(i, carry):
                for jj in range(k64 // 16):
                    sl = pl.ds(jj * 16, 16)
                    sv_[i, sl] = sv_[i, sl] + dv_[i, sl]
                return carry

            lax.fori_loop(0, ECH, cbody, 0)

            @pl.when(crow0 + j < RC)
            def _():
                off = pl.multiple_of((crow0 + j) * ECH, 8)
                pltpu.async_copy(sv_, out_h.at[pl.ds(off, ECH)], sw)

        def wait_store(j, sv_, sw):
            @pl.when(crow0 + j < RC)
            def _():
                off = pl.multiple_of((crow0 + j) * ECH, 8)
                pltpu.make_async_copy(sv_, out_h.at[pl.ds(off, ECH)], sw).wait()

        issue(0, sva, dva, ssa, sda)
        last_t = ENCH // 2 - 1

        def ebody(t, carry):
            c0 = 2 * t
            c1 = 2 * t + 1

            @pl.when(t > 0)
            def _():
                wait_store(c1 - 2, svb, swb)

            issue(c1, svb, dvb, ssb, sdb)
            wait(c0, sva, dva, ssa, sda)
            compute_store(c0, sva, dva, swa)

            @pl.when(t < last_t)
            def _():
                wait_store(c0, sva, swa)
                issue(c0 + 2, sva, dva, ssa, sda)

            wait(c1, svb, dvb, ssb, sdb)
            compute_store(c1, svb, dvb, swb)
            return carry

        lax.fori_loop(0, ENCH // 2, ebody, 0)
        wait_store(ENCH - 2, sva, swa)
        wait_store(ENCH - 1, svb, swb)

    return k(s_head, d_head, src, dst)


def _tc_dense(x, agg, wt, b, do_norm, do_relu, blk, row_off=0, mask_pad=False):
    bb, d = x.shape
    kk = wt.shape[1]
    has_agg = agg is not None
    nrows = bb - row_off * blk

    def body(*refs):
        xv = refs[0][...]
        if has_agg:
            av = refs[1][...]
            xv = xv + av[0] + av[1]
        iw = 1 + int(has_agg)
        if do_norm:
            s = jnp.sum(xv * xv, axis=1, keepdims=True)
            xv = xv / jnp.maximum(jnp.sqrt(s), 1e-12)
        y = jnp.dot(xv, refs[iw][...], preferred_element_type=jnp.float32)
        y = y + refs[iw + 1][...]
        if do_relu:
            y = jnp.maximum(y, 0.0)
        if mask_pad:
            rows = (jax.lax.broadcasted_iota(jnp.int32, (blk, kk), 0)
                    + pl.program_id(0) * blk)
            y = jnp.where(rows < N, y, -1e9)
        refs[iw + 2][...] = y

    in_specs = [pl.BlockSpec((blk, d), lambda i: (i + row_off, 0))]
    if has_agg:
        in_specs.append(pl.BlockSpec((NC, blk, d), lambda i: (0, i + row_off, 0)))
    in_specs += [
        pl.BlockSpec((d, kk), lambda i: (0, 0)),
        pl.BlockSpec((1, kk), lambda i: (0, 0)),
    ]
    args = (x, agg, wt, b) if has_agg else (x, wt, b)
    return pl.pallas_call(
        body,
        grid=(nrows // blk,),
        in_specs=in_specs,
        out_specs=pl.BlockSpec((blk, kk), lambda i: (i, 0)),
        out_shape=jax.ShapeDtypeStruct((nrows, kk), jnp.float32),
    )(*args)


def _tc_heads(x2, ws, bs, wd, wb, bb_):
    bb, d = x2.shape
    blk = 512

    def body(x_ref, ws_ref, bs_ref, wd_ref, wb_ref, bb_ref, s_ref, d_ref, b_ref):
        xv = x_ref[...]
        s_ref[...] = jnp.dot(xv, ws_ref[...], preferred_element_type=jnp.float32) + bs_ref[...]
        d_ref[...] = jnp.dot(xv, wd_ref[...], preferred_element_type=jnp.float32)
        b_ref[...] = jnp.dot(xv, wb_ref[...], preferred_element_type=jnp.float32) + bb_ref[...]

    return pl.pallas_call(
        body,
        grid=(bb // blk,),
        in_specs=[
            pl.BlockSpec((blk, d), lambda i: (i, 0)),
            pl.BlockSpec((d, 64), lambda i: (0, 0)),
            pl.BlockSpec((1, 64), lambda i: (0, 0)),
            pl.BlockSpec((d, 64), lambda i: (0, 0)),
            pl.BlockSpec((d, 8), lambda i: (0, 0)),
            pl.BlockSpec((1, 8), lambda i: (0, 0)),
        ],
        out_specs=[
            pl.BlockSpec((blk, 64), lambda i: (i, 0)),
            pl.BlockSpec((blk, 64), lambda i: (i, 0)),
            pl.BlockSpec((blk, 8), lambda i: (i, 0)),
        ],
        out_shape=[
            jax.ShapeDtypeStruct((bb, 64), jnp.float32),
            jax.ShapeDtypeStruct((bb, 64), jnp.float32),
            jax.ShapeDtypeStruct((bb, 8), jnp.float32),
        ],
    )(x2, ws, bs, wd, wb, bb_)


def kernel(node_ids, rel_ids, center_mol_idx, non_molecule_node_ids, edge_index,
           node_emb_table, rel_emb_table, lin_W, lin_b,
           conv1_W, conv1_b, conv2_W, conv2_b,
           ec_W, ec_b, mp_W, mp_b, nc_W, nc_b, bp_W, bp_b):
    f32 = jnp.float32
    i32 = jnp.int32
    node_ids = node_ids.astype(i32)
    rel_ids = rel_ids.astype(i32)
    src = edge_index[0].astype(i32)
    dst = edge_index[1].astype(i32)

    epw = E // NW
    ppw = E_PAD // NW - epw
    pad_s = jnp.full((NW, ppw), N, i32)
    src_p = jnp.concatenate([src.reshape(NW, epw), pad_s], 1).reshape(-1)
    rel_p = jnp.concatenate([rel_ids.reshape(NW, epw),
                             jnp.zeros((NW, ppw), i32)], 1).reshape(-1)
    pad_d = ((jnp.arange(NW * ppw, dtype=i32) * 1337) % NP).reshape(NW, ppw)
    dst_p = jnp.concatenate([dst.reshape(NW, epw), pad_d], 1).reshape(-1)
    npad = E_PAD - E
    src_q = jnp.concatenate([src, jnp.zeros((npad,), i32)])
    dst_q = jnp.concatenate([dst, jnp.zeros((npad,), i32)])

    ids_pad = jnp.concatenate([node_ids, jnp.zeros((NP - N,), i32)])
    rows = _sc_gather_rows(node_emb_table.astype(f32), ids_pad, 64)

    lin_bt = lin_b.reshape(1, -1)
    x0 = _tc_dense(rows, None, lin_W.T, lin_bt, True, True, 512, mask_pad=True)
    ea_u = _tc_dense(rel_emb_table.astype(f32), None, lin_W.T, lin_bt, True, True, 64)

    zblk = jnp.zeros((CH, D), f32)
    agg1 = _sc_msgpass(x0, ea_u, src_p, dst_p, rel_p, zblk)
    x1 = _tc_dense(x0, agg1, conv1_W.T, conv1_b.reshape(1, -1), False, True, 512,
                   mask_pad=True)
    agg2 = _sc_msgpass(x1, ea_u, src_p, dst_p, rel_p, zblk)
    x2 = _tc_dense(x1, agg2, conv2_W.T, conv2_b.reshape(1, -1), False, False, 512)

    wb = jnp.pad(bp_W.T, ((0, 0), (0, 7)))
    bb_ = jnp.pad(bp_b, (0, 7)).reshape(1, -1)
    s_head, d_head, bpad = _tc_heads(
        x2, ec_W[:, :D].T, ec_b.reshape(1, -1), ec_W[:, D:].T, wb, bb_)
    binary_pred = bpad[:N, :1]

    edge_class = _sc_edge_head(s_head, d_head, src_q, dst_q)

    gidx = jnp.concatenate([center_mol_idx.astype(i32),
                            non_molecule_node_ids.astype(i32)])
    xg = _sc_gather_rows(x2, gidx, 96)
    mp_wt = jnp.pad(mp_W.T, ((0, 0), (0, 28)))
    mp_bp = jnp.pad(mp_b, (0, 28)).reshape(1, -1)
    motif_pred = _tc_dense(xg, None, mp_wt, mp_bp, False, False, 512)[:1024, :100]
    nc_wt = jnp.pad(nc_W.T, ((0, 0), (0, 113)))
    nc_bp = jnp.pad(nc_b, (0, 113)).reshape(1, -1)
    node_class = _tc_dense(xg, None, nc_wt, nc_bp, False, False, 512, row_off=2)[:, :15]

    return (edge_class, motif_pred, node_class, binary_pred)
